# baseline (speedup 1.0000x reference)
import jax
import jax.numpy as jnp
from jax import lax
from jax.experimental import pallas as pl
from jax.experimental.pallas import tpu as pltpu

_N_CLUSTERS = 1024
_ITERS = 10
_DIM = 256
_ROW_BLOCK = 1024
_AUG = 3 * _DIM + 128


def _c2_row(c):
    ones_dim = jnp.ones((1, _DIM), jnp.float32)
    return lax.dot_general(ones_dim, c * c, (((1,), (1,)), ((), ())),
                           preferred_element_type=jnp.float32,
                           precision=lax.Precision.HIGHEST)


def _assign_block(xb, c, c2r):
    x2 = jnp.sum(xb * xb, axis=1, keepdims=True)
    xc = lax.dot_general(xb, c, (((1,), (1,)), ((), ())),
                         preferred_element_type=jnp.float32,
                         precision=lax.Precision.DEFAULT)
    d = x2 + c2r - 2.0 * xc
    return jnp.argmin(d, axis=1, keepdims=True).astype(jnp.int32)


def _split3_bf16(x):
    hi = x.astype(jnp.bfloat16)
    r1 = x - hi.astype(jnp.float32)
    mid = r1.astype(jnp.bfloat16)
    lo = (r1 - mid.astype(jnp.float32)).astype(jnp.bfloat16)
    return hi, mid, lo


def _km_kernel(x_ref, out_ref, c_ref, sums_ref, counts_ref, xaug_ref):
    n = x_ref.shape[0]
    nblk = n // _ROW_BLOCK
    c_ref[...] = x_ref[0:_N_CLUSTERS, :]

    def pre_blk(b, carry):
        sl = pl.ds(b * _ROW_BLOCK, _ROW_BLOCK)
        xb = x_ref[sl, :]
        hi, mid, lo = _split3_bf16(xb)
        xaug_ref[sl, 0:_DIM] = hi
        xaug_ref[sl, _DIM:2 * _DIM] = mid
        xaug_ref[sl, 2 * _DIM:3 * _DIM] = lo
        xaug_ref[sl, 3 * _DIM:_AUG] = jnp.ones(
            (_ROW_BLOCK, _AUG - 3 * _DIM), jnp.bfloat16)
        return carry

    lax.fori_loop(0, nblk, pre_blk, 0)

    def iter_body(it, carry):
        c = c_ref[...]
        c2r = _c2_row(c)
        sums_ref[...] = jnp.zeros_like(sums_ref)
        counts_ref[...] = jnp.zeros_like(counts_ref)

        def one_block(sl):
            assign = _assign_block(x_ref[sl, :], c, c2r)
            idx = lax.broadcasted_iota(jnp.int32, (_ROW_BLOCK, _N_CLUSTERS), 1)
            onehot = (assign == idx).astype(jnp.bfloat16)
            return lax.dot_general(onehot, xaug_ref[sl, :],
                                   (((0,), (0,)), ((), ())),
                                   preferred_element_type=jnp.float32)

        def blk(b, carry2):
            acc0 = one_block(pl.ds((2 * b) * _ROW_BLOCK, _ROW_BLOCK))
            acc1 = one_block(pl.ds((2 * b + 1) * _ROW_BLOCK, _ROW_BLOCK))
            acc = acc0 + acc1
            sums_ref[...] += (acc[:, 0:_DIM] + acc[:, _DIM:2 * _DIM]
                              + acc[:, 2 * _DIM:3 * _DIM])
            counts_ref[...] += acc[:, 3 * _DIM:3 * _DIM + 1]
            return carry2

        lax.fori_loop(0, nblk // 2, blk, 0)
        counts = counts_ref[...]
        newc = sums_ref[...] / jnp.maximum(counts, 1.0)
        c_ref[...] = jnp.where(counts > 0, newc, c)
        return carry

    lax.fori_loop(0, _ITERS, iter_body, 0)

    c = c_ref[...]
    c2r = _c2_row(c)

    def final_blk(b, carry):
        sl = pl.ds(b * _ROW_BLOCK, _ROW_BLOCK)
        assign = _assign_block(x_ref[sl, :], c, c2r)
        out_ref[sl] = jnp.squeeze(assign, -1)
        return carry

    lax.fori_loop(0, nblk, final_blk, 0)


def kernel(feat_g):
    n, dim = feat_g.shape
    out = pl.pallas_call(
        _km_kernel,
        out_shape=jax.ShapeDtypeStruct((n,), jnp.int32),
        scratch_shapes=[
            pltpu.VMEM((_N_CLUSTERS, _DIM), jnp.float32),
            pltpu.VMEM((_N_CLUSTERS, _DIM), jnp.float32),
            pltpu.VMEM((_N_CLUSTERS, 1), jnp.float32),
            pltpu.VMEM((n, _AUG), jnp.bfloat16),
        ],
        compiler_params=pltpu.CompilerParams(
            vmem_limit_bytes=63 * 1024 * 1024),
    )(feat_g)
    return out


# speedup vs baseline: 1.0499x; 1.0499x over previous
import jax
import jax.numpy as jnp
from jax import lax
from jax.experimental import pallas as pl
from jax.experimental.pallas import tpu as pltpu

_N_CLUSTERS = 1024
_ITERS = 10
_DIM = 256
_ROW_BLOCK = 2048
_AUG = 3 * _DIM + 128


def _c2_row(c):
    ones_dim = jnp.ones((1, _DIM), jnp.float32)
    return lax.dot_general(ones_dim, c * c, (((1,), (1,)), ((), ())),
                           preferred_element_type=jnp.float32,
                           precision=lax.Precision.HIGHEST)


def _assign_block(xb, c, c2r):
    x2 = jnp.sum(xb * xb, axis=1, keepdims=True)
    xc = lax.dot_general(xb, c, (((1,), (1,)), ((), ())),
                         preferred_element_type=jnp.float32,
                         precision=lax.Precision.DEFAULT)
    d = x2 + c2r - 2.0 * xc
    return jnp.argmin(d, axis=1, keepdims=True).astype(jnp.int32)


def _split3_bf16(x):
    hi = x.astype(jnp.bfloat16)
    r1 = x - hi.astype(jnp.float32)
    mid = r1.astype(jnp.bfloat16)
    lo = (r1 - mid.astype(jnp.float32)).astype(jnp.bfloat16)
    return hi, mid, lo


def _km_kernel(x_ref, out_ref, c_ref, sums_ref, counts_ref, xaug_ref):
    n = x_ref.shape[0]
    nblk = n // _ROW_BLOCK
    c_ref[...] = x_ref[0:_N_CLUSTERS, :]

    def pre_blk(b, carry):
        sl = pl.ds(b * _ROW_BLOCK, _ROW_BLOCK)
        xb = x_ref[sl, :]
        hi, mid, lo = _split3_bf16(xb)
        xaug_ref[sl, 0:_DIM] = hi
        xaug_ref[sl, _DIM:2 * _DIM] = mid
        xaug_ref[sl, 2 * _DIM:3 * _DIM] = lo
        xaug_ref[sl, 3 * _DIM:_AUG] = jnp.ones(
            (_ROW_BLOCK, _AUG - 3 * _DIM), jnp.bfloat16)
        return carry

    lax.fori_loop(0, nblk, pre_blk, 0)

    def iter_body(it, carry):
        c = c_ref[...]
        c2r = _c2_row(c)
        sums_ref[...] = jnp.zeros_like(sums_ref)
        counts_ref[...] = jnp.zeros_like(counts_ref)

        def one_block(sl):
            assign = _assign_block(x_ref[sl, :], c, c2r)
            idx = lax.broadcasted_iota(jnp.int32, (_ROW_BLOCK, _N_CLUSTERS), 1)
            onehot = (assign == idx).astype(jnp.bfloat16)
            return lax.dot_general(onehot, xaug_ref[sl, :],
                                   (((0,), (0,)), ((), ())),
                                   preferred_element_type=jnp.float32)

        def blk(b, carry2):
            acc = one_block(pl.ds(b * _ROW_BLOCK, _ROW_BLOCK))
            sums_ref[...] += (acc[:, 0:_DIM] + acc[:, _DIM:2 * _DIM]
                              + acc[:, 2 * _DIM:3 * _DIM])
            counts_ref[...] += acc[:, 3 * _DIM:3 * _DIM + 1]
            return carry2

        lax.fori_loop(0, nblk, blk, 0)
        counts = counts_ref[...]
        newc = sums_ref[...] / jnp.maximum(counts, 1.0)
        c_ref[...] = jnp.where(counts > 0, newc, c)
        return carry

    lax.fori_loop(0, _ITERS, iter_body, 0)

    c = c_ref[...]
    c2r = _c2_row(c)

    def final_blk(b, carry):
        sl = pl.ds(b * _ROW_BLOCK, _ROW_BLOCK)
        assign = _assign_block(x_ref[sl, :], c, c2r)
        out_ref[sl] = jnp.squeeze(assign, -1)
        return carry

    lax.fori_loop(0, nblk, final_blk, 0)


def kernel(feat_g):
    n, dim = feat_g.shape
    out = pl.pallas_call(
        _km_kernel,
        out_shape=jax.ShapeDtypeStruct((n,), jnp.int32),
        scratch_shapes=[
            pltpu.VMEM((_N_CLUSTERS, _DIM), jnp.float32),
            pltpu.VMEM((_N_CLUSTERS, _DIM), jnp.float32),
            pltpu.VMEM((_N_CLUSTERS, 1), jnp.float32),
            pltpu.VMEM((n, _AUG), jnp.bfloat16),
        ],
        compiler_params=pltpu.CompilerParams(
            vmem_limit_bytes=63 * 1024 * 1024),
    )(feat_g)
    return out


# speedup vs baseline: 1.0610x; 1.0105x over previous
import jax
import jax.numpy as jnp
from jax import lax
from jax.experimental import pallas as pl
from jax.experimental.pallas import tpu as pltpu

_N_CLUSTERS = 1024
_ITERS = 10
_DIM = 256
_ROW_BLOCK = 2048


def _c2_row(c):
    ones_dim = jnp.ones((1, _DIM), jnp.float32)
    return lax.dot_general(ones_dim, c * c, (((1,), (1,)), ((), ())),
                           preferred_element_type=jnp.float32,
                           precision=lax.Precision.HIGHEST)


def _assign_block(xb, c, c2r):
    x2 = jnp.sum(xb * xb, axis=1, keepdims=True)
    xc = lax.dot_general(xb, c, (((1,), (1,)), ((), ())),
                         preferred_element_type=jnp.float32,
                         precision=lax.Precision.DEFAULT)
    d = x2 + c2r - 2.0 * xc
    return jnp.argmin(d, axis=1, keepdims=True).astype(jnp.int32)


def _split3_bf16(x):
    hi = x.astype(jnp.bfloat16)
    r1 = x - hi.astype(jnp.float32)
    mid = r1.astype(jnp.bfloat16)
    lo = (r1 - mid.astype(jnp.float32)).astype(jnp.bfloat16)
    return hi, mid, lo


def _km_kernel(x_ref, out_ref, c_ref, sums_ref, counts_ref,
               xhi_ref, xmid_ref, xlo_ref):
    n = x_ref.shape[0]
    nblk = n // _ROW_BLOCK
    c_ref[...] = x_ref[0:_N_CLUSTERS, :]
    ones_col = jnp.ones((_ROW_BLOCK, 1), jnp.bfloat16)

    def pre_blk(b, carry):
        sl = pl.ds(b * _ROW_BLOCK, _ROW_BLOCK)
        xb = x_ref[sl, :]
        hi, mid, lo = _split3_bf16(xb)
        xhi_ref[sl, :] = hi
        xmid_ref[sl, :] = mid
        xlo_ref[sl, :] = lo
        return carry

    lax.fori_loop(0, nblk, pre_blk, 0)

    def iter_body(it, carry):
        c = c_ref[...]
        c2r = _c2_row(c)
        sums_ref[...] = jnp.zeros_like(sums_ref)
        counts_ref[...] = jnp.zeros_like(counts_ref)

        def blk(b, carry2):
            sl = pl.ds(b * _ROW_BLOCK, _ROW_BLOCK)
            assign = _assign_block(x_ref[sl, :], c, c2r)
            idx = lax.broadcasted_iota(jnp.int32, (_ROW_BLOCK, _N_CLUSTERS), 1)
            onehot = (assign == idx).astype(jnp.bfloat16)
            cdims = (((0,), (0,)), ((), ()))
            acc = lax.dot_general(onehot, xhi_ref[sl, :], cdims,
                                  preferred_element_type=jnp.float32)
            acc += lax.dot_general(onehot, xmid_ref[sl, :], cdims,
                                   preferred_element_type=jnp.float32)
            acc += lax.dot_general(onehot, xlo_ref[sl, :], cdims,
                                   preferred_element_type=jnp.float32)
            sums_ref[...] += acc
            counts_ref[...] += lax.dot_general(
                onehot, ones_col, cdims,
                preferred_element_type=jnp.float32)
            return carry2

        lax.fori_loop(0, nblk, blk, 0)
        counts = counts_ref[...]
        newc = sums_ref[...] / jnp.maximum(counts, 1.0)
        c_ref[...] = jnp.where(counts > 0, newc, c)
        return carry

    lax.fori_loop(0, _ITERS, iter_body, 0)

    c = c_ref[...]
    c2r = _c2_row(c)

    def final_blk(b, carry):
        sl = pl.ds(b * _ROW_BLOCK, _ROW_BLOCK)
        out_ref[sl, :] = _assign_block(x_ref[sl, :], c, c2r)
        return carry

    lax.fori_loop(0, nblk, final_blk, 0)


def kernel(feat_g):
    n, dim = feat_g.shape
    preds = pl.pallas_call(
        _km_kernel,
        out_shape=jax.ShapeDtypeStruct((n, 1), jnp.int32),
        scratch_shapes=[
            pltpu.VMEM((_N_CLUSTERS, _DIM), jnp.float32),
            pltpu.VMEM((_N_CLUSTERS, _DIM), jnp.float32),
            pltpu.VMEM((_N_CLUSTERS, 1), jnp.float32),
            pltpu.VMEM((n, _DIM), jnp.bfloat16),
            pltpu.VMEM((n, _DIM), jnp.bfloat16),
            pltpu.VMEM((n, _DIM), jnp.bfloat16),
        ],
        compiler_params=pltpu.CompilerParams(
            vmem_limit_bytes=100 * 1024 * 1024),
    )(feat_g)
    return preds.reshape(n)
